# baseline (speedup 1.0000x reference)
import jax
import jax.numpy as jnp
from jax.experimental import pallas as pl


_ROWS = 768
_COLS = 50176
_BM = 32


def _relu_block(x_ref, o_ref):
    o_ref[...] = jnp.maximum(x_ref[...], 0.0)


def kernel(x):
    x2 = x.reshape(_ROWS, _COLS)
    out = pl.pallas_call(
        _relu_block,
        grid=(_ROWS // _BM,),
        in_specs=[pl.BlockSpec((_BM, _COLS), lambda i: (i, 0))],
        out_specs=pl.BlockSpec((_BM, _COLS), lambda i: (i, 0)),
        out_shape=jax.ShapeDtypeStruct((_ROWS, _COLS), x.dtype),
    )(x2)
    return out.reshape(x.shape)
